# baseline (speedup 1.0000x reference)
import functools

import jax
import jax.numpy as jnp
from jax import lax
from jax.experimental import pallas as pl
from jax.experimental.pallas import tpu as pltpu
from jax.experimental.pallas import tpu_sc as plsc

DIM = 128
NUM_CORES = 2
NUM_SUBCORES = 16
NUM_WORKERS = NUM_CORES * NUM_SUBCORES
CHUNK = 160
NBUF = 4


def kernel(x, weight):
    b0, b1 = x.shape
    num_idx = b0 * b1
    idx = x.astype(jnp.int32).T.reshape(num_idx)
    per_worker = num_idx // NUM_WORKERS
    n_chunks = per_worker // CHUNK
    n_groups = n_chunks // NBUF

    mesh = plsc.VectorSubcoreMesh(core_axis_name="c", subcore_axis_name="s")

    @functools.partial(
        pl.kernel,
        mesh=mesh,
        out_type=jax.ShapeDtypeStruct((num_idx, DIM), jnp.float32),
        scratch_types=[
            pltpu.VMEM((per_worker,), jnp.int32),
            pltpu.VMEM((NBUF, CHUNK, DIM), jnp.float32),
            pltpu.SemaphoreType.DMA((NBUF,)),
        ],
    )
    def gather_kernel(table_hbm, idx_hbm, out_hbm, idx_v, rows_v, sems):
        wid = lax.axis_index("s") * NUM_CORES + lax.axis_index("c")
        base = wid * per_worker

        def gather_desc(i, b):
            return pltpu.make_async_copy(
                table_hbm.at[idx_v.at[pl.ds(i * CHUNK, CHUNK)]],
                rows_v.at[b],
                sems.at[b],
            )

        def store(i, b):
            pltpu.sync_copy(rows_v.at[b], out_hbm.at[pl.ds(base + i * CHUNK, CHUNK)])

        pltpu.sync_copy(idx_hbm.at[pl.ds(base, per_worker)], idx_v)

        for b in range(NBUF):
            gather_desc(b, b).start()

        @pl.loop(0, n_groups - 1)
        def _(g):
            for b in range(NBUF):
                i = g * NBUF + b
                gather_desc(i, b).wait()
                store(i, b)
                gather_desc(i + NBUF, b).start()

        for b in range(NBUF):
            i = (n_groups - 1) * NBUF + b
            gather_desc(i, b).wait()
            store(i, b)

    out = gather_kernel(weight, idx)
    return out.reshape(b1, b0, DIM).transpose(1, 0, 2)


# speedup vs baseline: 1.0094x; 1.0094x over previous
import functools

import jax
import jax.numpy as jnp
from jax import lax
from jax.experimental import pallas as pl
from jax.experimental.pallas import tpu as pltpu
from jax.experimental.pallas import tpu_sc as plsc

DIM = 128
NUM_CORES = 2
NUM_SUBCORES = 16
NUM_WORKERS = NUM_CORES * NUM_SUBCORES
CHUNK = 160
NBUF = 4


def kernel(x, weight):
    b0, b1 = x.shape
    num_idx = b0 * b1
    idx = x.astype(jnp.int32).T.reshape(num_idx)
    per_worker = num_idx // NUM_WORKERS
    n_chunks = per_worker // CHUNK
    n_groups = n_chunks // NBUF

    mesh = plsc.VectorSubcoreMesh(core_axis_name="c", subcore_axis_name="s")

    @functools.partial(
        pl.kernel,
        mesh=mesh,
        out_type=jax.ShapeDtypeStruct((num_idx, DIM), jnp.float32),
        scratch_types=[
            pltpu.VMEM((per_worker,), jnp.int32),
            pltpu.VMEM((NBUF, CHUNK, DIM), jnp.float32),
            pltpu.VMEM_SHARED((NUM_SUBCORES, 2, CHUNK, DIM), jnp.float32),
            pltpu.SemaphoreType.DMA((NBUF,)),
            pltpu.SemaphoreType.DMA((2,)),
        ],
    )
    def gather_kernel(table_hbm, idx_hbm, out_hbm, idx_v, rows_v, sp, gsems, ssems):
        wid = lax.axis_index("s") * NUM_CORES + lax.axis_index("c")
        tid = lax.axis_index("s")
        base = wid * per_worker

        def gather_desc(i, b):
            return pltpu.make_async_copy(
                table_hbm.at[idx_v.at[pl.ds(i * CHUNK, CHUNK)]],
                rows_v.at[b],
                gsems.at[b],
            )

        def store_desc(i, s):
            return pltpu.make_async_copy(
                sp.at[tid, s],
                out_hbm.at[pl.ds(base + i * CHUNK, CHUNK)],
                ssems.at[s],
            )

        pltpu.sync_copy(idx_hbm.at[pl.ds(base, per_worker)], idx_v)

        for b in range(NBUF):
            gather_desc(b, b).start()

        def step(i, b, refill, drain):
            s = b % 2
            gather_desc(i, b).wait()
            if drain:
                store_desc(i - 2, s).wait()
            pltpu.sync_copy(rows_v.at[b], sp.at[tid, s])
            if refill:
                gather_desc(i + NBUF, b).start()
            store_desc(i, s).start()

        for b in range(NBUF):
            step(b, b, refill=True, drain=b >= 2)

        @pl.loop(1, n_groups - 1)
        def _(g):
            for b in range(NBUF):
                step(g * NBUF + b, b, refill=True, drain=True)

        last = (n_groups - 1) * NBUF
        for b in range(NBUF):
            step(last + b, b, refill=False, drain=True)
        store_desc(last + 2, 0).wait()
        store_desc(last + 3, 1).wait()

    out = gather_kernel(weight, idx)
    return out.reshape(b1, b0, DIM).transpose(1, 0, 2)
